# Initial kernel scaffold; baseline (speedup 1.0000x reference)
import jax
import jax.numpy as jnp
from jax.experimental import pallas as pl


def kernel(nodes_ft, adj_list, weight, bias, att_layer_1, att_layer_2):
    raise NotImplementedError("write your pallas kernel here")


# speedup vs baseline: 3.2432x; 3.2432x over previous
import functools

import jax
import jax.numpy as jnp
from jax import lax
from jax.experimental import pallas as pl
from jax.experimental.pallas import tpu as pltpu
from jax.experimental.pallas import tpu_sc as plsc

NS = 16
NC = 2
L = 16
DH = 64


def _tc_tables_body(x_ref, w_ref, a1_ref, a2_ref, ai2_ref, jh_ref):
    h = jnp.dot(x_ref[...], w_ref[...], preferred_element_type=jnp.float32)
    ai = jnp.dot(h, a1_ref[...], preferred_element_type=jnp.float32)
    aj = jnp.dot(h, a2_ref[...], preferred_element_type=jnp.float32)
    ai2_ref[0] = ai[:, :DH]
    ai2_ref[1] = ai[:, DH:]
    jh_ref[0] = jnp.concatenate([aj[:, :DH], h[:, :DH]], axis=1)
    jh_ref[1] = jnp.concatenate([aj[:, DH:], h[:, DH:]], axis=1)


def _tc_tables(x, w, a1, a2):
    n, d = x.shape
    bn = 1000
    assert n % bn == 0
    return pl.pallas_call(
        _tc_tables_body,
        grid=(n // bn,),
        in_specs=[
            pl.BlockSpec((bn, d), lambda i: (i, 0)),
            pl.BlockSpec((d, d), lambda i: (0, 0)),
            pl.BlockSpec((d, d), lambda i: (0, 0)),
            pl.BlockSpec((d, d), lambda i: (0, 0)),
        ],
        out_specs=[
            pl.BlockSpec((NC, bn, DH), lambda i: (0, i, 0)),
            pl.BlockSpec((NC, bn, 2 * DH), lambda i: (0, i, 0)),
        ],
        out_shape=[
            jax.ShapeDtypeStruct((NC, n, DH), jnp.float32),
            jax.ShapeDtypeStruct((NC, n, 2 * DH), jnp.float32),
        ],
    )(x, w, a1, a2)


def _make_sc_edge_kernel(n, e, npad):
    K = 80
    EC = e // NS
    NCHUNK = EC // K
    RCH = 80
    RPT = npad // NS
    assert EC * NS == e and NCHUNK * K == EC
    assert RPT % RCH == 0 and RPT % 8 == 0 and npad >= n

    mesh = plsc.VectorSubcoreMesh(core_axis_name="c", subcore_axis_name="s")

    @functools.partial(
        pl.kernel,
        out_type=jax.ShapeDtypeStruct((NC * npad, DH), jnp.float32),
        mesh=mesh,
        scratch_types=[
            pltpu.VMEM((K,), jnp.int32),
            pltpu.VMEM((K,), jnp.int32),
            pltpu.VMEM((K,), jnp.int32),
            pltpu.VMEM((K, DH), jnp.float32),
            pltpu.VMEM((K, 2 * DH), jnp.float32),
            pltpu.VMEM((K, 2 * DH), jnp.float32),
            pltpu.VMEM_SHARED((npad, 2 * DH), jnp.float32),
            pltpu.VMEM((RCH, 2 * DH), jnp.float32),
            pltpu.VMEM((RCH, DH), jnp.float32),
            pltpu.VMEM((DH,), jnp.float32),
            pltpu.SemaphoreType.DMA,
            pltpu.SemaphoreType.DMA,
        ],
        compiler_params=pltpu.CompilerParams(use_tc_tiling_on_sc=False),
    )
    def sc_edge(ai_hbm, jh_hbm, src_hbm, dst_hbm, bias_hbm, out_hbm,
                srcv, srcg, dstg, abuf, jhbuf, vpbuf, numden, ndbuf, obuf,
                biasv, sem1, sem2):
        c = lax.axis_index("c")
        s = lax.axis_index("s")
        cn = c * n
        cnp = c * npad

        zero = jnp.zeros((L,), jnp.float32)

        def zrow(r, _):
            for g in range(2 * DH // L):
                ndbuf[r, pl.ds(g * L, L)] = zero
            return 0

        lax.fori_loop(0, RCH, zrow, 0)

        def zchunk(jr, _):
            pltpu.sync_copy(ndbuf, numden.at[pl.ds(s * RPT + jr * RCH, RCH)])
            return 0

        lax.fori_loop(0, RPT // RCH, zchunk, 0)
        plsc.subcore_barrier()

        ebase = s * EC

        def chunk(j, _):
            base = ebase + j * K
            pltpu.sync_copy(src_hbm.at[pl.ds(base, K)], srcv)
            pltpu.sync_copy(dst_hbm.at[pl.ds(base, K)], dstg)

            def addcn(i, _):
                srcg[pl.ds(i * L, L)] = srcv[pl.ds(i * L, L)] + cn
                dstg[pl.ds(i * L, L)] = dstg[pl.ds(i * L, L)] + cn
                return 0

            lax.fori_loop(0, K // L, addcn, 0)
            cp1 = pltpu.async_copy(ai_hbm.at[srcg], abuf, sem1)
            cp2 = pltpu.async_copy(jh_hbm.at[dstg], jhbuf, sem2)
            cp1.wait()
            cp2.wait()

            def edge(ei, _):
                for g in range(DH // L):
                    a = abuf[ei, pl.ds(g * L, L)]
                    b = jhbuf[ei, pl.ds(g * L, L)]
                    hh = jhbuf[ei, pl.ds(DH + g * L, L)]
                    lg = a + b
                    lg = jnp.maximum(lg, 0.2 * lg)
                    p = jnp.exp(lg)
                    vpbuf[ei, pl.ds(g * L, L)] = p * hh
                    vpbuf[ei, pl.ds(DH + g * L, L)] = p
                return 0

            lax.fori_loop(0, K, edge, 0)
            pltpu.sync_copy(vpbuf, numden.at[srcv], add=True)
            return 0

        lax.fori_loop(0, NCHUNK, chunk, 0)
        plsc.subcore_barrier()

        pltpu.sync_copy(bias_hbm.at[pl.ds(c * DH, DH)], biasv)

        def rchunk(jr, _):
            r0 = s * RPT + jr * RCH
            pltpu.sync_copy(numden.at[pl.ds(r0, RCH)], ndbuf)

            def row(r, _):
                for g in range(DH // L):
                    nm = ndbuf[r, pl.ds(g * L, L)]
                    dn = ndbuf[r, pl.ds(DH + g * L, L)]
                    bv = biasv[pl.ds(g * L, L)]
                    obuf[r, pl.ds(g * L, L)] = nm / (dn + 1e-16) + bv
                return 0

            lax.fori_loop(0, RCH, row, 0)
            pltpu.sync_copy(obuf, out_hbm.at[pl.ds(cnp + r0, RCH)])
            return 0

        lax.fori_loop(0, RPT // RCH, rchunk, 0)

    return sc_edge


def kernel(nodes_ft, adj_list, weight, bias, att_layer_1, att_layer_2):
    n, d = nodes_ft.shape
    e = adj_list.shape[1]
    npad = ((n + 2559) // 2560) * 2560
    ai2, jh = _tc_tables(nodes_ft, weight, att_layer_1, att_layer_2)
    sc_edge = _make_sc_edge_kernel(n, e, npad)
    out2 = sc_edge(
        ai2.reshape(NC * n, DH),
        jh.reshape(NC * n, 2 * DH),
        adj_list[0],
        adj_list[1],
        bias,
    )
    return out2.reshape(NC, npad, DH)[:, :n].transpose(1, 0, 2).reshape(n, d)


# speedup vs baseline: 4.0364x; 1.2446x over previous
import functools

import jax
import jax.numpy as jnp
from jax import lax
from jax.experimental import pallas as pl
from jax.experimental.pallas import tpu as pltpu
from jax.experimental.pallas import tpu_sc as plsc

NS = 16
NC = 2
L = 16
DH = 64


def _tc_tables_body(x_ref, w_ref, a1_ref, a2_ref, ai2_ref, jh_ref):
    h = jnp.dot(x_ref[...], w_ref[...], preferred_element_type=jnp.float32)
    ai = jnp.dot(h, a1_ref[...], preferred_element_type=jnp.float32)
    aj = jnp.dot(h, a2_ref[...], preferred_element_type=jnp.float32)
    ai2_ref[0] = ai[:, :DH]
    ai2_ref[1] = ai[:, DH:]
    jh_ref[0] = jnp.concatenate([aj[:, :DH], h[:, :DH]], axis=1)
    jh_ref[1] = jnp.concatenate([aj[:, DH:], h[:, DH:]], axis=1)


def _tc_tables(x, w, a1, a2):
    n, d = x.shape
    bn = 1000
    assert n % bn == 0
    return pl.pallas_call(
        _tc_tables_body,
        grid=(n // bn,),
        in_specs=[
            pl.BlockSpec((bn, d), lambda i: (i, 0)),
            pl.BlockSpec((d, d), lambda i: (0, 0)),
            pl.BlockSpec((d, d), lambda i: (0, 0)),
            pl.BlockSpec((d, d), lambda i: (0, 0)),
        ],
        out_specs=[
            pl.BlockSpec((NC, bn, DH), lambda i: (0, i, 0)),
            pl.BlockSpec((NC, bn, 2 * DH), lambda i: (0, i, 0)),
        ],
        out_shape=[
            jax.ShapeDtypeStruct((NC, n, DH), jnp.float32),
            jax.ShapeDtypeStruct((NC, n, 2 * DH), jnp.float32),
        ],
    )(x, w, a1, a2)


def _make_sc_edge_kernel(n, e, npad):
    K = 80
    NSUB = 10
    SCH = NSUB * K
    EC = e // NS
    NSUPER = EC // SCH
    RCH = 8
    RPT = npad // NS
    assert EC * NS == e and NSUPER * SCH == EC and NSUB % 2 == 0
    assert RPT % RCH == 0 and RPT % 8 == 0 and npad >= n

    mesh = plsc.VectorSubcoreMesh(core_axis_name="c", subcore_axis_name="s")

    @functools.partial(
        pl.kernel,
        out_type=jax.ShapeDtypeStruct((NC * npad, DH), jnp.float32),
        mesh=mesh,
        scratch_types=[
            pltpu.VMEM((NSUB, K), jnp.int32),
            pltpu.VMEM((NSUB, K), jnp.int32),
            pltpu.VMEM((NSUB, K), jnp.int32),
            pltpu.VMEM((2, K, DH), jnp.float32),
            pltpu.VMEM((2, K, 2 * DH), jnp.float32),
            pltpu.VMEM((K, 2 * DH), jnp.float32),
            pltpu.VMEM_SHARED((npad, 2 * DH), jnp.float32),
            pltpu.VMEM((RCH, 2 * DH), jnp.float32),
            pltpu.VMEM((RCH, DH), jnp.float32),
            pltpu.VMEM((DH,), jnp.float32),
            pltpu.SemaphoreType.DMA,
            pltpu.SemaphoreType.DMA,
            pltpu.SemaphoreType.DMA,
            pltpu.SemaphoreType.DMA,
            pltpu.SemaphoreType.DMA,
        ],
        compiler_params=pltpu.CompilerParams(use_tc_tiling_on_sc=False),
    )
    def sc_edge(src_hbm, dst_hbm, ai_hbm, jh_hbm, bias_hbm, out_hbm,
                srcv2d, srcg2d, dstg2d, abuf, jhbuf, vpbuf, numden,
                ndbuf, obuf, biasv, sema0, sema1, semj0, semj1, semi):
        c = lax.axis_index("c")
        s = lax.axis_index("s")
        cn = c * n
        cnp = c * npad
        sem_a = (sema0, sema1)
        sem_j = (semj0, semj1)

        zero = jnp.zeros((L,), jnp.float32)

        def zrow(r, _):
            for g in range(2 * DH // L):
                ndbuf[r, pl.ds(g * L, L)] = zero
            return 0

        lax.fori_loop(0, RCH, zrow, 0)

        def zchunk(jr, _):
            pltpu.sync_copy(ndbuf, numden.at[pl.ds(s * RPT + jr * RCH, RCH)])
            return 0

        lax.fori_loop(0, RPT // RCH, zchunk, 0)
        plsc.subcore_barrier()

        ebase = s * EC

        def fire(k, buf):
            cpa = pltpu.async_copy(ai_hbm.at[srcg2d.at[k]], abuf.at[buf],
                                   sem_a[buf])
            cpj = pltpu.async_copy(jh_hbm.at[dstg2d.at[k]], jhbuf.at[buf],
                                   sem_j[buf])
            return cpa, cpj

        def compute_scatter(k, buf):
            pltpu.make_async_copy(ai_hbm.at[srcg2d.at[k]], abuf.at[buf],
                                  sem_a[buf]).wait()
            pltpu.make_async_copy(jh_hbm.at[dstg2d.at[k]], jhbuf.at[buf],
                                  sem_j[buf]).wait()

            def edge(ei, _):
                for g in range(DH // L):
                    a = abuf[buf, ei, pl.ds(g * L, L)]
                    b = jhbuf[buf, ei, pl.ds(g * L, L)]
                    hh = jhbuf[buf, ei, pl.ds(DH + g * L, L)]
                    lg = a + b
                    lg = jnp.maximum(lg, 0.2 * lg)
                    p = jnp.exp(lg)
                    vpbuf[ei, pl.ds(g * L, L)] = p * hh
                    vpbuf[ei, pl.ds(DH + g * L, L)] = p
                return 0

            lax.fori_loop(0, K, edge, 0, unroll=2)
            pltpu.sync_copy(vpbuf, numden.at[srcv2d.at[k]], add=True)

        def superchunk(sc_i, _):
            base = ebase + sc_i * SCH
            for i in range(NSUB):
                pltpu.async_copy(src_hbm.at[pl.ds(base + i * K, K)],
                                 srcv2d.at[i], semi)
                pltpu.async_copy(dst_hbm.at[pl.ds(base + i * K, K)],
                                 dstg2d.at[i], semi)
            for i in range(NSUB):
                pltpu.make_async_copy(src_hbm.at[pl.ds(base + i * K, K)],
                                      srcv2d.at[i], semi).wait()
                pltpu.make_async_copy(dst_hbm.at[pl.ds(base + i * K, K)],
                                      dstg2d.at[i], semi).wait()

            def addcn(i, _):
                for q in range(K // L):
                    sl = pl.ds(q * L, L)
                    srcg2d[i, sl] = srcv2d[i, sl] + cn
                    dstg2d[i, sl] = dstg2d[i, sl] + cn
                return 0

            lax.fori_loop(0, NSUB, addcn, 0)

            fire(0, 0)

            def pipe(k2, _):
                k = 2 * k2
                fire(k + 1, 1)
                compute_scatter(k, 0)

                @pl.when(k + 2 < NSUB)
                def _():
                    fire(k + 2, 0)

                compute_scatter(k + 1, 1)
                return 0

            lax.fori_loop(0, NSUB // 2, pipe, 0)
            return 0

        lax.fori_loop(0, NSUPER, superchunk, 0)
        plsc.subcore_barrier()

        pltpu.sync_copy(bias_hbm.at[pl.ds(c * DH, DH)], biasv)

        def rchunk(jr, _):
            r0 = s * RPT + jr * RCH
            pltpu.sync_copy(numden.at[pl.ds(r0, RCH)], ndbuf)

            def row(r, _):
                for g in range(DH // L):
                    nm = ndbuf[r, pl.ds(g * L, L)]
                    dn = ndbuf[r, pl.ds(DH + g * L, L)]
                    bv = biasv[pl.ds(g * L, L)]
                    obuf[r, pl.ds(g * L, L)] = nm / (dn + 1e-16) + bv
                return 0

            lax.fori_loop(0, RCH, row, 0)
            pltpu.sync_copy(obuf, out_hbm.at[pl.ds(cnp + r0, RCH)])
            return 0

        lax.fori_loop(0, RPT // RCH, rchunk, 0)

    return sc_edge


def kernel(nodes_ft, adj_list, weight, bias, att_layer_1, att_layer_2):
    n, d = nodes_ft.shape
    e = adj_list.shape[1]
    npad = ((n + 127) // 128) * 128
    ai2, jh = _tc_tables(nodes_ft, weight, att_layer_1, att_layer_2)
    sc_edge = _make_sc_edge_kernel(n, e, npad)
    out2 = sc_edge(
        adj_list[0],
        adj_list[1],
        ai2.reshape(NC * n, DH),
        jh.reshape(NC * n, 2 * DH),
        bias,
    )
    return out2.reshape(NC, npad, DH)[:, :n].transpose(1, 0, 2).reshape(n, d)


# speedup vs baseline: 11.4614x; 2.8395x over previous
import functools

import jax
import jax.numpy as jnp
from jax import lax
from jax.experimental import pallas as pl
from jax.experimental.pallas import tpu as pltpu
from jax.experimental.pallas import tpu_sc as plsc

NS = 16
NC = 2
L = 16
DH = 64


def _tc_tables_body(x_ref, w_ref, a1_ref, a2_ref, ai2_ref, jh_ref):
    h = jnp.dot(x_ref[...], w_ref[...], preferred_element_type=jnp.float32)
    ai = jnp.dot(h, a1_ref[...], preferred_element_type=jnp.float32)
    aj = jnp.dot(h, a2_ref[...], preferred_element_type=jnp.float32)
    ai2_ref[0] = ai[:, :DH]
    ai2_ref[1] = ai[:, DH:]
    jh_ref[0] = jnp.concatenate([aj[:, :DH], h[:, :DH]], axis=1)
    jh_ref[1] = jnp.concatenate([aj[:, DH:], h[:, DH:]], axis=1)


def _tc_tables(x, w, a1, a2):
    n, d = x.shape
    bn = 1000
    assert n % bn == 0
    return pl.pallas_call(
        _tc_tables_body,
        grid=(n // bn,),
        in_specs=[
            pl.BlockSpec((bn, d), lambda i: (i, 0)),
            pl.BlockSpec((d, d), lambda i: (0, 0)),
            pl.BlockSpec((d, d), lambda i: (0, 0)),
            pl.BlockSpec((d, d), lambda i: (0, 0)),
        ],
        out_specs=[
            pl.BlockSpec((NC, bn, DH), lambda i: (0, i, 0)),
            pl.BlockSpec((NC, bn, 2 * DH), lambda i: (0, i, 0)),
        ],
        out_shape=[
            jax.ShapeDtypeStruct((NC, n, DH), jnp.float32),
            jax.ShapeDtypeStruct((NC, n, 2 * DH), jnp.float32),
        ],
    )(x, w, a1, a2)


def _make_sc_edge_kernel(n, e, npad):
    K = 80
    NSUB = 10
    SCH = NSUB * K
    EC = e // NS
    NSUPER = EC // SCH
    RCH = 8
    RPT = npad // NS
    assert EC * NS == e and NSUPER * SCH == EC and NSUB % 2 == 0
    assert RPT % RCH == 0 and RPT % 8 == 0 and npad >= n

    mesh = plsc.VectorSubcoreMesh(core_axis_name="c", subcore_axis_name="s")

    @functools.partial(
        pl.kernel,
        out_type=jax.ShapeDtypeStruct((NC * npad, DH), jnp.float32),
        mesh=mesh,
        scratch_types=[
            pltpu.VMEM((NSUB, K), jnp.int32),
            pltpu.VMEM((NSUB, K), jnp.int32),
            pltpu.VMEM((NSUB, K), jnp.int32),
            pltpu.VMEM((2, K, DH), jnp.float32),
            pltpu.VMEM((2, K, 2 * DH), jnp.float32),
            pltpu.VMEM((K, 2 * DH), jnp.float32),
            pltpu.VMEM_SHARED((npad, 2 * DH), jnp.float32),
            pltpu.VMEM((RCH, 2 * DH), jnp.float32),
            pltpu.VMEM((RCH, DH), jnp.float32),
            pltpu.VMEM((DH,), jnp.float32),
            pltpu.SemaphoreType.DMA,
            pltpu.SemaphoreType.DMA,
            pltpu.SemaphoreType.DMA,
            pltpu.SemaphoreType.DMA,
            pltpu.SemaphoreType.DMA,
        ],
        compiler_params=pltpu.CompilerParams(use_tc_tiling_on_sc=False),
    )
    def sc_edge(src_hbm, dst_hbm, ai_hbm, jh_hbm, bias_hbm, out_hbm,
                srcv2d, srcg2d, dstg2d, abuf, jhbuf, vpbuf, numden,
                ndbuf, obuf, biasv, sema0, sema1, semj0, semj1, semi):
        c = lax.axis_index("c")
        s = lax.axis_index("s")
        cn = c * n
        cnp = c * npad
        sem_a = (sema0, sema1)
        sem_j = (semj0, semj1)

        zero = jnp.zeros((L,), jnp.float32)

        def zrow(r, _):
            for g in range(2 * DH // L):
                ndbuf[r, pl.ds(g * L, L)] = zero
            return 0

        lax.fori_loop(0, RCH, zrow, 0)

        def zchunk(jr, _):
            pltpu.sync_copy(ndbuf, numden.at[pl.ds(s * RPT + jr * RCH, RCH)])
            return 0

        lax.fori_loop(0, RPT // RCH, zchunk, 0)
        plsc.subcore_barrier()

        ebase = s * EC

        def fire(k, buf):
            cpa = pltpu.async_copy(ai_hbm.at[srcg2d.at[k]], abuf.at[buf],
                                   sem_a[buf])
            cpj = pltpu.async_copy(jh_hbm.at[dstg2d.at[k]], jhbuf.at[buf],
                                   sem_j[buf])
            return cpa, cpj

        def compute_scatter(k, buf):
            pltpu.make_async_copy(ai_hbm.at[srcg2d.at[k]], abuf.at[buf],
                                  sem_a[buf]).wait()
            pltpu.make_async_copy(jh_hbm.at[dstg2d.at[k]], jhbuf.at[buf],
                                  sem_j[buf]).wait()

            @plsc.parallel_loop(0, K, 1, unroll=4)
            def edge(ei):
                for g in range(DH // L):
                    a = abuf[buf, ei, pl.ds(g * L, L)]
                    b = jhbuf[buf, ei, pl.ds(g * L, L)]
                    hh = jhbuf[buf, ei, pl.ds(DH + g * L, L)]
                    lg = a + b
                    lg = jnp.maximum(lg, 0.2 * lg)
                    p = jnp.exp(lg)
                    vpbuf[ei, pl.ds(g * L, L)] = p * hh
                    vpbuf[ei, pl.ds(DH + g * L, L)] = p

            pltpu.sync_copy(vpbuf, numden.at[srcv2d.at[k]], add=True)

        def superchunk(sc_i, _):
            base = ebase + sc_i * SCH
            for i in range(NSUB):
                pltpu.async_copy(src_hbm.at[pl.ds(base + i * K, K)],
                                 srcv2d.at[i], semi)
                pltpu.async_copy(dst_hbm.at[pl.ds(base + i * K, K)],
                                 dstg2d.at[i], semi)
            for i in range(NSUB):
                pltpu.make_async_copy(src_hbm.at[pl.ds(base + i * K, K)],
                                      srcv2d.at[i], semi).wait()
                pltpu.make_async_copy(dst_hbm.at[pl.ds(base + i * K, K)],
                                      dstg2d.at[i], semi).wait()

            def addcn(i, _):
                for q in range(K // L):
                    sl = pl.ds(q * L, L)
                    srcg2d[i, sl] = srcv2d[i, sl] + cn
                    dstg2d[i, sl] = dstg2d[i, sl] + cn
                return 0

            lax.fori_loop(0, NSUB, addcn, 0)

            fire(0, 0)

            def pipe(k2, _):
                k = 2 * k2
                fire(k + 1, 1)
                compute_scatter(k, 0)

                @pl.when(k + 2 < NSUB)
                def _():
                    fire(k + 2, 0)

                compute_scatter(k + 1, 1)
                return 0

            lax.fori_loop(0, NSUB // 2, pipe, 0)
            return 0

        lax.fori_loop(0, NSUPER, superchunk, 0)
        plsc.subcore_barrier()

        pltpu.sync_copy(bias_hbm.at[pl.ds(c * DH, DH)], biasv)

        def rchunk(jr, _):
            r0 = s * RPT + jr * RCH
            pltpu.sync_copy(numden.at[pl.ds(r0, RCH)], ndbuf)

            def row(r, _):
                for g in range(DH // L):
                    nm = ndbuf[r, pl.ds(g * L, L)]
                    dn = ndbuf[r, pl.ds(DH + g * L, L)]
                    bv = biasv[pl.ds(g * L, L)]
                    obuf[r, pl.ds(g * L, L)] = nm / (dn + 1e-16) + bv
                return 0

            lax.fori_loop(0, RCH, row, 0)
            pltpu.sync_copy(obuf, out_hbm.at[pl.ds(cnp + r0, RCH)])
            return 0

        lax.fori_loop(0, RPT // RCH, rchunk, 0)

    return sc_edge


def kernel(nodes_ft, adj_list, weight, bias, att_layer_1, att_layer_2):
    n, d = nodes_ft.shape
    e = adj_list.shape[1]
    npad = ((n + 127) // 128) * 128
    ai2, jh = _tc_tables(nodes_ft, weight, att_layer_1, att_layer_2)
    sc_edge = _make_sc_edge_kernel(n, e, npad)
    out2 = sc_edge(
        adj_list[0],
        adj_list[1],
        ai2.reshape(NC * n, DH),
        jh.reshape(NC * n, 2 * DH),
        bias,
    )
    return out2.reshape(NC, npad, DH)[:, :n].transpose(1, 0, 2).reshape(n, d)


# speedup vs baseline: 12.3864x; 1.0807x over previous
import functools

import jax
import jax.numpy as jnp
from jax import lax
from jax.experimental import pallas as pl
from jax.experimental.pallas import tpu as pltpu
from jax.experimental.pallas import tpu_sc as plsc

NS = 16
NC = 2
L = 16
DH = 64


def _tc_tables_body(x_ref, w_ref, a1_ref, a2_ref, ai2_ref, jh_ref):
    h = jnp.dot(x_ref[...], w_ref[...], preferred_element_type=jnp.float32)
    ai = jnp.dot(h, a1_ref[...], preferred_element_type=jnp.float32)
    aj = jnp.dot(h, a2_ref[...], preferred_element_type=jnp.float32)
    ai2_ref[0] = ai[:, :DH]
    ai2_ref[1] = ai[:, DH:]
    jh_ref[0] = jnp.concatenate([aj[:, :DH], h[:, :DH]], axis=1)
    jh_ref[1] = jnp.concatenate([aj[:, DH:], h[:, DH:]], axis=1)


def _tc_tables(x, w, a1, a2):
    n, d = x.shape
    bn = 1000
    assert n % bn == 0
    return pl.pallas_call(
        _tc_tables_body,
        grid=(n // bn,),
        in_specs=[
            pl.BlockSpec((bn, d), lambda i: (i, 0)),
            pl.BlockSpec((d, d), lambda i: (0, 0)),
            pl.BlockSpec((d, d), lambda i: (0, 0)),
            pl.BlockSpec((d, d), lambda i: (0, 0)),
        ],
        out_specs=[
            pl.BlockSpec((NC, bn, DH), lambda i: (0, i, 0)),
            pl.BlockSpec((NC, bn, 2 * DH), lambda i: (0, i, 0)),
        ],
        out_shape=[
            jax.ShapeDtypeStruct((NC, n, DH), jnp.float32),
            jax.ShapeDtypeStruct((NC, n, 2 * DH), jnp.float32),
        ],
    )(x, w, a1, a2)


def _make_sc_edge_kernel(n, e, npad):
    K = 80
    NSUB = 10
    SCH = NSUB * K
    EC = e // NS
    NSUPER = EC // SCH
    RCH = 8
    RPT = npad // NS
    assert EC * NS == e and NSUPER * SCH == EC and NSUB % 2 == 0
    assert RPT % RCH == 0 and RPT % 8 == 0 and npad >= n

    mesh = plsc.VectorSubcoreMesh(core_axis_name="c", subcore_axis_name="s")

    @functools.partial(
        pl.kernel,
        out_type=jax.ShapeDtypeStruct((NC * npad, DH), jnp.float32),
        mesh=mesh,
        scratch_types=[
            pltpu.VMEM((NSUB, K), jnp.int32),
            pltpu.VMEM((NSUB, K), jnp.int32),
            pltpu.VMEM((NSUB, K), jnp.int32),
            pltpu.VMEM((2, K, DH), jnp.float32),
            pltpu.VMEM((2, K, 2 * DH), jnp.float32),
            pltpu.VMEM((K, 2 * DH), jnp.float32),
            pltpu.VMEM_SHARED((npad, 2 * DH), jnp.float32),
            pltpu.VMEM((RCH, 2 * DH), jnp.float32),
            pltpu.VMEM((RCH, DH), jnp.float32),
            pltpu.VMEM((DH,), jnp.float32),
            pltpu.SemaphoreType.DMA,
            pltpu.SemaphoreType.DMA,
            pltpu.SemaphoreType.DMA,
            pltpu.SemaphoreType.DMA,
            pltpu.SemaphoreType.DMA,
            pltpu.SemaphoreType.DMA,
        ],
        compiler_params=pltpu.CompilerParams(use_tc_tiling_on_sc=False),
    )
    def sc_edge(src_hbm, dst_hbm, ai_hbm, jh_hbm, bias_hbm, out_hbm,
                srcv2d, srcg2d, dstg2d, abuf, jhbuf, vpbuf, numden,
                ndbuf, obuf, biasv, sema0, sema1, semj0, semj1, semi, semv):
        c = lax.axis_index("c")
        s = lax.axis_index("s")
        cn = c * n
        cnp = c * npad
        sem_a = (sema0, sema1)
        sem_j = (semj0, semj1)

        zero = jnp.zeros((L,), jnp.float32)

        def zrow(r, _):
            for g in range(2 * DH // L):
                ndbuf[r, pl.ds(g * L, L)] = zero
            return 0

        lax.fori_loop(0, RCH, zrow, 0)

        def zchunk(jr, _):
            pltpu.sync_copy(ndbuf, numden.at[pl.ds(s * RPT + jr * RCH, RCH)])
            return 0

        lax.fori_loop(0, RPT // RCH, zchunk, 0)
        plsc.subcore_barrier()

        ebase = s * EC

        def fire(k, buf):
            cpa = pltpu.async_copy(ai_hbm.at[srcg2d.at[k]], abuf.at[buf],
                                   sem_a[buf])
            cpj = pltpu.async_copy(jh_hbm.at[dstg2d.at[k]], jhbuf.at[buf],
                                   sem_j[buf])
            return cpa, cpj

        def wait_gathers(k, buf):
            pltpu.make_async_copy(ai_hbm.at[srcg2d.at[k]], abuf.at[buf],
                                  sem_a[buf]).wait()
            pltpu.make_async_copy(jh_hbm.at[dstg2d.at[k]], jhbuf.at[buf],
                                  sem_j[buf]).wait()

        def wait_scatter():
            pltpu.make_async_copy(vpbuf, numden.at[srcv2d.at[0]],
                                  semv).wait()

        def compute(k, buf):
            @plsc.parallel_loop(0, K, 1, unroll=8)
            def edge(ei):
                for g in range(DH // L):
                    a = abuf[buf, ei, pl.ds(g * L, L)]
                    b = jhbuf[buf, ei, pl.ds(g * L, L)]
                    hh = jhbuf[buf, ei, pl.ds(DH + g * L, L)]
                    lg = a + b
                    lg = jnp.maximum(lg, 0.2 * lg)
                    p = jnp.exp(lg)
                    vpbuf[ei, pl.ds(g * L, L)] = p * hh
                    vpbuf[ei, pl.ds(DH + g * L, L)] = p

        def fire_scatter(k):
            pltpu.async_copy(vpbuf, numden.at[srcv2d.at[k]], semv, add=True)

        def superchunk(sc_i, _):
            @pl.when(sc_i > 0)
            def _():
                wait_scatter()

            base = ebase + sc_i * SCH
            for i in range(NSUB):
                pltpu.async_copy(src_hbm.at[pl.ds(base + i * K, K)],
                                 srcv2d.at[i], semi)
                pltpu.async_copy(dst_hbm.at[pl.ds(base + i * K, K)],
                                 dstg2d.at[i], semi)
            for i in range(NSUB):
                pltpu.make_async_copy(src_hbm.at[pl.ds(base + i * K, K)],
                                      srcv2d.at[i], semi).wait()
                pltpu.make_async_copy(dst_hbm.at[pl.ds(base + i * K, K)],
                                      dstg2d.at[i], semi).wait()

            def addcn(i, _):
                for q in range(K // L):
                    sl = pl.ds(q * L, L)
                    srcg2d[i, sl] = srcv2d[i, sl] + cn
                    dstg2d[i, sl] = dstg2d[i, sl] + cn
                return 0

            lax.fori_loop(0, NSUB, addcn, 0)

            fire(0, 0)

            def pipe(k2, _):
                k = 2 * k2
                fire(k + 1, 1)
                wait_gathers(k, 0)

                @pl.when(k2 > 0)
                def _():
                    wait_scatter()

                compute(k, 0)
                fire_scatter(k)

                @pl.when(k + 2 < NSUB)
                def _():
                    fire(k + 2, 0)

                wait_gathers(k + 1, 1)
                wait_scatter()
                compute(k + 1, 1)
                fire_scatter(k + 1)
                return 0

            lax.fori_loop(0, NSUB // 2, pipe, 0)
            return 0

        lax.fori_loop(0, NSUPER, superchunk, 0)
        wait_scatter()
        plsc.subcore_barrier()

        pltpu.sync_copy(bias_hbm.at[pl.ds(c * DH, DH)], biasv)

        def rchunk(jr, _):
            r0 = s * RPT + jr * RCH
            pltpu.sync_copy(numden.at[pl.ds(r0, RCH)], ndbuf)

            def row(r, _):
                for g in range(DH // L):
                    nm = ndbuf[r, pl.ds(g * L, L)]
                    dn = ndbuf[r, pl.ds(DH + g * L, L)]
                    bv = biasv[pl.ds(g * L, L)]
                    obuf[r, pl.ds(g * L, L)] = nm / (dn + 1e-16) + bv
                return 0

            lax.fori_loop(0, RCH, row, 0)
            pltpu.sync_copy(obuf, out_hbm.at[pl.ds(cnp + r0, RCH)])
            return 0

        lax.fori_loop(0, RPT // RCH, rchunk, 0)

    return sc_edge


def kernel(nodes_ft, adj_list, weight, bias, att_layer_1, att_layer_2):
    n, d = nodes_ft.shape
    e = adj_list.shape[1]
    npad = ((n + 127) // 128) * 128
    ai2, jh = _tc_tables(nodes_ft, weight, att_layer_1, att_layer_2)
    sc_edge = _make_sc_edge_kernel(n, e, npad)
    out2 = sc_edge(
        adj_list[0],
        adj_list[1],
        ai2.reshape(NC * n, DH),
        jh.reshape(NC * n, 2 * DH),
        bias,
    )
    return out2.reshape(NC, npad, DH)[:, :n].transpose(1, 0, 2).reshape(n, d)
